# baseline (speedup 1.0000x reference)
import functools

import jax
import jax.numpy as jnp
from jax import lax
from jax.experimental import pallas as pl
from jax.experimental.pallas import tpu as pltpu
from jax.experimental.pallas import tpu_sc as plsc

_NW = 32


def _route_topk2(scores_w, n, b):
    pts = b // _NW
    ngrp = pts // 16
    nc = 2
    mesh = plsc.VectorSubcoreMesh(core_axis_name="c", subcore_axis_name="s")

    @functools.partial(
        pl.kernel, mesh=mesh,
        out_type=(jax.ShapeDtypeStruct((b,), jnp.int32),
                  jax.ShapeDtypeStruct((b,), jnp.int32)),
        scratch_types=[
            pltpu.VMEM((n, pts), jnp.float32),
            pltpu.VMEM((pts,), jnp.int32),
            pltpu.VMEM((pts,), jnp.int32),
        ],
    )
    def k(scores_hbm, i1_hbm, i2_hbm, s_v, i1_v, i2_v):
        wid = lax.axis_index("s") * nc + lax.axis_index("c")
        base = wid * pts
        pltpu.sync_copy(scores_hbm.at[wid], s_v)

        neg = jnp.full((16,), jnp.int32(-2147483648), jnp.int32)
        zero = jnp.zeros((16,), jnp.int32)
        for g in range(ngrp):
            def body(e, carry):
                m1, i1, m2, i2 = carry
                v = s_v[e, pl.ds(g * 16, 16)]
                vb = jax.lax.bitcast_convert_type(v, jnp.int32)
                key = jnp.where(vb < 0, vb ^ jnp.int32(0x7FFFFFFF), vb)
                ev = jnp.full((16,), e, jnp.int32)
                gt1 = key > m1
                gt2 = key > m2
                m2n = jnp.where(gt1, m1, jnp.where(gt2, key, m2))
                i2n = jnp.where(gt1, i1, jnp.where(gt2, ev, i2))
                m1n = jnp.where(gt1, key, m1)
                i1n = jnp.where(gt1, ev, i1)
                return m1n, i1n, m2n, i2n

            _, i1, _, i2 = lax.fori_loop(0, n, body, (neg, zero, neg, zero))
            i1_v[pl.ds(g * 16, 16)] = i1
            i2_v[pl.ds(g * 16, 16)] = i2

        pltpu.sync_copy(i1_v, i1_hbm.at[pl.ds(base, pts)])
        pltpu.sync_copy(i2_v, i2_hbm.at[pl.ds(base, pts)])

    return k(scores_w)


def _expert_step(i1_ref, i2_ref, x_ref,
                 w1, b1, w2, b2, w3, b3, w4, b4,
                 cw, cb, dwd, rw, rb,
                 out_ref, *, cxyz, whalf):
    e = pl.program_id(0)
    xyz = x_ref[:, :cxyz]
    xdir = x_ref[:, cxyz:]
    f32 = jnp.float32

    def mm(a, wref):
        return jnp.dot(a, wref[0], preferred_element_type=f32)

    h = jax.nn.relu(mm(xyz, w1) + b1[0])
    h = jax.nn.relu(mm(h, w2) + b2[0])
    h = jax.nn.relu(mm(h, w3) + b3[0])
    h = jax.nn.relu(mm(h, w4) + b4[0])

    t = mm(h, cw) + cb[0]
    dire = jax.nn.relu(t[:, :whalf] + mm(xdir, dwd))
    sigma = jax.nn.softplus(t[:, whalf:])
    rgb = jax.nn.sigmoid(mm(dire, rw) + rb[0])

    contrib = jnp.concatenate([rgb, sigma], axis=1)
    sel0 = jnp.where(i1_ref[...] == e, contrib, out_ref[:, :4])
    sel1 = jnp.where(i2_ref[...] == e, contrib, out_ref[:, 4:])
    out_ref[...] = jnp.concatenate([sel0, sel1], axis=1)


def _moe_mlp(i1, i2, x, ws, n, b, cxyz, whalf):
    (w1, b1, w2, b2, w3, b3, w4, b4, cw, cb, dwd, rw, rb) = ws
    w = w1.shape[2]

    def wspec(i, o):
        return pl.BlockSpec((1, i, o), lambda e: (e, 0, 0))

    def bspec(o):
        return pl.BlockSpec((1, 1, o), lambda e: (e, 0, 0))

    def full2(arr):
        return pl.BlockSpec(arr.shape, lambda e: (0, 0))

    out8 = pl.pallas_call(
        functools.partial(_expert_step, cxyz=cxyz, whalf=whalf),
        grid=(n,),
        in_specs=[
            full2(i1), full2(i2), full2(x),
            wspec(cxyz, w), bspec(w),
            wspec(w, w), bspec(w),
            wspec(w, w), bspec(w),
            wspec(w, w), bspec(w),
            wspec(w, whalf + 1), bspec(whalf + 1),
            wspec(x.shape[1] - cxyz, whalf),
            wspec(whalf, rw.shape[2]), bspec(rw.shape[2]),
        ],
        out_specs=pl.BlockSpec((b, 8), lambda e: (0, 0)),
        out_shape=jax.ShapeDtypeStruct((b, 8), jnp.float32),
        compiler_params=pltpu.CompilerParams(
            dimension_semantics=("arbitrary",),
        ),
    )(i1, i2, x, w1, b1, w2, b2, w3, b3, w4, b4,
      cw, cb, dwd, rw, rb)
    return jnp.transpose(out8.reshape(b, 2, 4), (1, 0, 2))


def kernel(x, rbfs, xyz_w_1, xyz_b_1, xyz_w_2, xyz_b_2, xyz_w_3, xyz_b_3,
           xyz_w_4, xyz_b_4, final_w, final_b, dir_w, dir_b, sigma_w,
           sigma_b, rgb_w, rgb_b):
    n, b, _ = rbfs.shape
    cxyz = xyz_w_1.shape[1]

    scores_w = jnp.transpose(
        jnp.squeeze(rbfs, -1).reshape(n, _NW, b // _NW), (1, 0, 2))
    i1, i2 = _route_topk2(scores_w, n, b)

    whalf = dir_w.shape[2]
    w = final_w.shape[2]
    dwh = dir_w[:, :w, :]
    dwd = dir_w[:, w:, :]
    cw = jnp.concatenate(
        [jnp.einsum('nio,nod->nid', final_w, dwh), sigma_w], axis=2)
    cb = jnp.concatenate(
        [jnp.einsum('no,nod->nd', final_b, dwh) + dir_b,
         sigma_b - 1.0], axis=1)

    (xyz_b_1, xyz_b_2, xyz_b_3, xyz_b_4, cb, rgb_b) = (
        a[:, None, :] for a in (xyz_b_1, xyz_b_2, xyz_b_3, xyz_b_4,
                                cb, rgb_b))

    ws = (xyz_w_1, xyz_b_1, xyz_w_2, xyz_b_2, xyz_w_3, xyz_b_3,
          xyz_w_4, xyz_b_4, cw, cb, dwd, rgb_w, rgb_b)
    return _moe_mlp(i1.reshape(b, 1), i2.reshape(b, 1), x, ws, n, b,
                    cxyz, whalf)


# speedup vs baseline: 1.1627x; 1.1627x over previous
import functools

import jax
import jax.numpy as jnp
from jax import lax
from jax.experimental import pallas as pl
from jax.experimental.pallas import tpu as pltpu
from jax.experimental.pallas import tpu_sc as plsc

_NW = 32


def _route_topk2(scores_w, n, b):
    pts = b // _NW
    ngrp = pts // 16
    nc = 2
    mesh = plsc.VectorSubcoreMesh(core_axis_name="c", subcore_axis_name="s")

    @functools.partial(
        pl.kernel, mesh=mesh,
        out_type=(jax.ShapeDtypeStruct((b,), jnp.int32),
                  jax.ShapeDtypeStruct((b,), jnp.int32)),
        scratch_types=[
            pltpu.VMEM((n, pts), jnp.float32),
            pltpu.VMEM((pts,), jnp.int32),
            pltpu.VMEM((pts,), jnp.int32),
        ],
    )
    def k(scores_hbm, i1_hbm, i2_hbm, s_v, i1_v, i2_v):
        wid = lax.axis_index("s") * nc + lax.axis_index("c")
        base = wid * pts
        pltpu.sync_copy(scores_hbm.at[wid], s_v)

        neg = jnp.full((16,), jnp.int32(-2147483648), jnp.int32)
        zero = jnp.zeros((16,), jnp.int32)
        for g in range(ngrp):
            def body(e, carry):
                m1, i1, m2, i2 = carry
                v = s_v[e, pl.ds(g * 16, 16)]
                vb = jax.lax.bitcast_convert_type(v, jnp.int32)
                key = jnp.where(vb < 0, vb ^ jnp.int32(0x7FFFFFFF), vb)
                ev = jnp.full((16,), e, jnp.int32)
                gt1 = key > m1
                gt2 = key > m2
                m2n = jnp.where(gt1, m1, jnp.where(gt2, key, m2))
                i2n = jnp.where(gt1, i1, jnp.where(gt2, ev, i2))
                m1n = jnp.where(gt1, key, m1)
                i1n = jnp.where(gt1, ev, i1)
                return m1n, i1n, m2n, i2n

            _, i1, _, i2 = lax.fori_loop(0, n, body, (neg, zero, neg, zero))
            i1_v[pl.ds(g * 16, 16)] = i1
            i2_v[pl.ds(g * 16, 16)] = i2

        pltpu.sync_copy(i1_v, i1_hbm.at[pl.ds(base, pts)])
        pltpu.sync_copy(i2_v, i2_hbm.at[pl.ds(base, pts)])

    return k(scores_w)


def _expert_step(i1_ref, i2_ref, x_ref,
                 w1, b1, w2, b2, w3, b3, w4, b4,
                 fw, fb, dw, db, sw, sb, rw, rb,
                 out_ref, *, cxyz):
    e = pl.program_id(0)
    xyz = x_ref[:, :cxyz]
    xdir = x_ref[:, cxyz:]
    f32 = jnp.float32

    def mm(a, wref):
        return jnp.dot(a, wref[0], preferred_element_type=f32)

    h = jax.nn.relu(mm(xyz, w1) + b1[0])
    h = jax.nn.relu(mm(h, w2) + b2[0])
    h = jax.nn.relu(mm(h, w3) + b3[0])
    h = jax.nn.relu(mm(h, w4) + b4[0])

    hdim = fw.shape[1]
    dwh = dw[0][:hdim, :]
    cwe = jnp.dot(fw[0], dwh, preferred_element_type=f32)
    cbe = jnp.dot(fb[0], dwh, preferred_element_type=f32) + db[0]
    dire = jax.nn.relu(
        jnp.dot(h, cwe, preferred_element_type=f32)
        + jnp.dot(xdir, dw[0][hdim:, :], preferred_element_type=f32)
        + cbe)
    sigma = jax.nn.softplus(mm(h, sw) + sb[0] - 1.0)
    rgb = jax.nn.sigmoid(mm(dire, rw) + rb[0])

    contrib = jnp.concatenate([rgb, sigma], axis=1)
    out_ref[0] = jnp.where(i1_ref[...] == e, contrib, out_ref[0])
    out_ref[1] = jnp.where(i2_ref[...] == e, contrib, out_ref[1])


def _moe_mlp(i1, i2, x, ws, n, b, cxyz):
    (w1, b1, w2, b2, w3, b3, w4, b4, fw, fb, dw, db, sw, sb, rw, rb) = ws
    w = w1.shape[2]
    whalf = dw.shape[2]

    def wspec(i, o):
        return pl.BlockSpec((1, i, o), lambda e: (e, 0, 0))

    def bspec(o):
        return pl.BlockSpec((1, 1, o), lambda e: (e, 0, 0))

    def full2(arr):
        return pl.BlockSpec(arr.shape, lambda e: (0, 0))

    return pl.pallas_call(
        functools.partial(_expert_step, cxyz=cxyz),
        grid=(n,),
        in_specs=[
            full2(i1), full2(i2), full2(x),
            wspec(cxyz, w), bspec(w),
            wspec(w, w), bspec(w),
            wspec(w, w), bspec(w),
            wspec(w, w), bspec(w),
            wspec(w, w), bspec(w),
            wspec(w + x.shape[1] - cxyz, whalf), bspec(whalf),
            wspec(w, 1), bspec(1),
            wspec(whalf, rw.shape[2]), bspec(rw.shape[2]),
        ],
        out_specs=pl.BlockSpec((2, b, 4), lambda e: (0, 0, 0)),
        out_shape=jax.ShapeDtypeStruct((2, b, 4), jnp.float32),
        compiler_params=pltpu.CompilerParams(
            dimension_semantics=("arbitrary",),
        ),
    )(i1, i2, x, w1, b1, w2, b2, w3, b3, w4, b4,
      fw, fb, dw, db, sw, sb, rw, rb)


def kernel(x, rbfs, xyz_w_1, xyz_b_1, xyz_w_2, xyz_b_2, xyz_w_3, xyz_b_3,
           xyz_w_4, xyz_b_4, final_w, final_b, dir_w, dir_b, sigma_w,
           sigma_b, rgb_w, rgb_b):
    n, b, _ = rbfs.shape
    cxyz = xyz_w_1.shape[1]

    scores_w = jnp.transpose(
        jnp.squeeze(rbfs, -1).reshape(n, _NW, b // _NW), (1, 0, 2))
    i1, i2 = _route_topk2(scores_w, n, b)

    (xyz_b_1, xyz_b_2, xyz_b_3, xyz_b_4, final_b, dir_b, sigma_b, rgb_b) = (
        a[:, None, :] for a in (xyz_b_1, xyz_b_2, xyz_b_3, xyz_b_4,
                                final_b, dir_b, sigma_b, rgb_b))

    ws = (xyz_w_1, xyz_b_1, xyz_w_2, xyz_b_2, xyz_w_3, xyz_b_3,
          xyz_w_4, xyz_b_4, final_w, final_b, dir_w, dir_b,
          sigma_w, sigma_b, rgb_w, rgb_b)
    return _moe_mlp(i1.reshape(b, 1), i2.reshape(b, 1), x, ws, n, b, cxyz)
